# Initial kernel scaffold; baseline (speedup 1.0000x reference)
import jax
import jax.numpy as jnp
from jax.experimental import pallas as pl


def kernel(x, adj1):
    raise NotImplementedError("write your pallas kernel here")


# speedup vs baseline: 1.9594x; 1.9594x over previous
import functools

import jax
import jax.numpy as jnp
from jax import lax
from jax.experimental import pallas as pl
from jax.experimental.pallas import tpu as pltpu
from jax.experimental.pallas import tpu_sc as plsc

N = 10000
E = 160000
D = 256
K = 3

NC = 2
NQ = 2
NS = 16
DQ = D // (NC * NQ)
ET = E // NS
CH = 128
NBUF = 4
NCH = (ET + CH - 1) // CH
NCH = ((NCH + NBUF - 1) // NBUF) * NBUF
NP = 10240
RPT = NP // NS


def _make_kernel():
    mesh = plsc.VectorSubcoreMesh(core_axis_name="c", subcore_axis_name="s")

    @functools.partial(
        pl.kernel,
        out_type=jax.ShapeDtypeStruct((NC * NQ * NP, DQ), jnp.float32),
        mesh=mesh,
        compiler_params=pltpu.CompilerParams(use_tc_tiling_on_sc=False),
        scratch_types=[
            pltpu.VMEM((NQ, NCH + NBUF, CH), jnp.int32),
            pltpu.VMEM((NCH, CH), jnp.int32),
            pltpu.VMEM((CH, DQ), jnp.float32),
            pltpu.VMEM((CH, DQ), jnp.float32),
            pltpu.VMEM((CH, DQ), jnp.float32),
            pltpu.VMEM((CH, DQ), jnp.float32),
            pltpu.VMEM_SHARED((NP, DQ), jnp.float32),
            pltpu.SemaphoreType.DMA,
            pltpu.SemaphoreType.DMA,
            pltpu.SemaphoreType.DMA,
            pltpu.SemaphoreType.DMA,
        ],
    )
    def kgin_sc(xr_hbm, colb_hbm, rowb_hbm, zrows_hbm, out_hbm,
                colv, rowv, b0, b1, b2, b3, acc, s0, s1, s2, s3):
        c = lax.axis_index("c")
        s = lax.axis_index("s")
        wid = c * NS + s
        bufs = (b0, b1, b2, b3)
        sems = (s0, s1, s2, s3)

        pltpu.sync_copy(colb_hbm.at[wid], colv)
        pltpu.sync_copy(rowb_hbm.at[s], rowv)

        def qpass(src, q):
            cq = colv.at[q]
            pltpu.sync_copy(zrows_hbm, acc.at[pl.ds(s * RPT, RPT)])
            plsc.subcore_barrier()

            for b in range(NBUF):
                pltpu.async_copy(src.at[cq.at[b]], bufs[b], sems[b])

            def outer(i, carry):
                j0 = i * NBUF
                for b in range(NBUF):
                    j = j0 + b
                    pltpu.make_async_copy(src.at[cq.at[j]], bufs[b],
                                          sems[b]).wait()
                    pltpu.sync_copy(bufs[b], acc.at[rowv.at[j]], add=True)
                    pltpu.async_copy(src.at[cq.at[j + NBUF]], bufs[b],
                                     sems[b])
                return carry

            lax.fori_loop(0, NCH // NBUF, outer, 0)

            for b in range(NBUF):
                pltpu.make_async_copy(src.at[cq.at[0]], bufs[b],
                                      sems[b]).wait()

            plsc.subcore_barrier()
            pltpu.sync_copy(acc.at[pl.ds(s * RPT, RPT)],
                            out_hbm.at[pl.ds((c * NQ + q) * NP + s * RPT,
                                             RPT)])
            plsc.subcore_barrier()

        def hop(src):
            for q in range(NQ):
                qpass(src, q)

        hop(xr_hbm)
        for _ in range(K - 1):
            hop(out_hbm)

    return kgin_sc


_KGIN = _make_kernel()


def kernel(x, adj1):
    row = adj1[0].astype(jnp.int32)
    col = adj1[1].astype(jnp.int32)

    pad = NCH * CH - ET
    col_t = jnp.pad(col.reshape(NS, ET), ((0, 0), (0, pad)))
    row_t = jnp.pad(row.reshape(NS, ET), ((0, 0), (0, pad)),
                    constant_values=N)
    col_t = col_t.reshape(NS, NCH, CH)
    row_t = row_t.reshape(NS, NCH, CH)

    col_pad = jnp.concatenate(
        [col_t, jnp.zeros((NS, NBUF, CH), jnp.int32)], axis=1)
    colb = jnp.stack(
        [jnp.stack([col_pad + (ci * NQ + qi) * NP for qi in range(NQ)])
         for ci in range(NC)])
    colb = colb.transpose(0, 2, 1, 3, 4).reshape(
        NC * NS, NQ, NCH + NBUF, CH)

    xp = jnp.pad(x, ((0, NP - N), (0, 0)))
    xr = xp.reshape(NP, NC * NQ, DQ).transpose(1, 0, 2).reshape(
        NC * NQ * NP, DQ)
    zrows = jnp.zeros((RPT, DQ), jnp.float32)

    out_r = _KGIN(xr, colb, row_t, zrows)
    return out_r.reshape(NC * NQ, NP, DQ)[:, :N].transpose(1, 0, 2).reshape(
        N, D)
